# baseline (speedup 1.0000x reference)
import functools

import jax
import jax.numpy as jnp
from jax import lax
from jax.experimental import pallas as pl
from jax.experimental.pallas import tpu as pltpu
from jax.experimental.pallas import tpu_sc as plsc

EMBED = 64
LANES = 16
NC = 2
NS = 16
NW = NC * NS

BLK = 1024
G = BLK // 128
HALF = BLK // 2
GH = G // 2


def _build(ntok):
    rows_per_w = ntok // NW
    nblk = rows_per_w // BLK
    mesh = plsc.VectorSubcoreMesh(core_axis_name="c", subcore_axis_name="s")

    @functools.partial(
        pl.kernel,
        out_type=jax.ShapeDtypeStruct((ntok, EMBED), jnp.float32),
        mesh=mesh,
        compiler_params=pltpu.CompilerParams(use_tc_tiling_on_sc=False),
        scratch_types=[
            pltpu.VMEM((G, 128), jnp.int32),
            pltpu.VMEM((G, 128), jnp.int32),
            pltpu.VMEM((HALF, EMBED), jnp.float32),
            pltpu.VMEM((HALF, EMBED), jnp.float32),
            pltpu.SemaphoreType.DMA,
            pltpu.SemaphoreType.DMA,
        ],
    )
    def emb_kernel(tok_hbm, pidx_hbm, table_hbm, pos_hbm, out_hbm,
                   tok_v, pidx_v, rows_v, pos_rows_v, sem_a, sem_b):
        wid = lax.axis_index("s") * NC + lax.axis_index("c")
        w_base = wid * rows_per_w

        def blk_body(ch, carry):
            base = w_base + ch * BLK
            idx_row0 = pl.multiple_of(base // 128, 8)
            pltpu.sync_copy(tok_hbm.at[pl.ds(idx_row0, G)], tok_v)
            pltpu.sync_copy(pidx_hbm.at[pl.ds(idx_row0, G)], pidx_v)
            for h in range(2):
                copies = []
                for j in range(GH):
                    copies.append(pltpu.async_copy(
                        table_hbm.at[tok_v.at[h * GH + j]],
                        rows_v.at[pl.ds(j * 128, 128)], sem_a))
                    copies.append(pltpu.async_copy(
                        pos_hbm.at[pidx_v.at[h * GH + j]],
                        pos_rows_v.at[pl.ds(j * 128, 128)], sem_b))
                for cp in copies:
                    cp.wait()

                @plsc.parallel_loop(0, HALF, unroll=8)
                def _row_body(r):
                    for k in range(EMBED // LANES):
                        sl = pl.ds(k * LANES, LANES)
                        rows_v[r, sl] = rows_v[r, sl] + pos_rows_v[r, sl]
                pltpu.sync_copy(rows_v, out_hbm.at[pl.ds(base + h * HALF, HALF)])
            return carry

        lax.fori_loop(0, nblk, blk_body, 0)

    return emb_kernel


def kernel(x, padding_mask, table, pos_enc):
    b, s = x.shape
    ntok = b * s
    tok = x.reshape(ntok // 128, 128).astype(jnp.int32)
    s_ids = jnp.arange(s, dtype=jnp.int32)[None, :]
    pidx = jnp.where(padding_mask, jnp.int32(s), s_ids)
    pidx = pidx.reshape(ntok // 128, 128).astype(jnp.int32)
    pos_ext = jnp.concatenate(
        [pos_enc.astype(jnp.float32),
         jnp.zeros((1, pos_enc.shape[1]), jnp.float32)], axis=0)
    out = _build(ntok)(tok, pidx, table, pos_ext)
    return out.reshape(b, s, EMBED)


# speedup vs baseline: 5.6444x; 5.6444x over previous
import functools

import jax
import jax.numpy as jnp
from jax import lax
from jax.experimental import pallas as pl
from jax.experimental.pallas import tpu as pltpu
from jax.experimental.pallas import tpu_sc as plsc

EMBED = 64
LANES = 16
NC = 2
NS = 16
NW = NC * NS

BLK = 1024
HALF = BLK // 2
POS_REP = 64


def _build(ntok):
    rows_per_w = ntok // NW
    nblk = rows_per_w // BLK
    mesh = plsc.VectorSubcoreMesh(core_axis_name="c", subcore_axis_name="s")

    @functools.partial(
        pl.kernel,
        out_type=jax.ShapeDtypeStruct((ntok, EMBED), jnp.float32),
        mesh=mesh,
        compiler_params=pltpu.CompilerParams(use_tc_tiling_on_sc=False),
        scratch_types=[
            pltpu.VMEM((BLK,), jnp.int32),
            pltpu.VMEM((BLK,), jnp.int32),
            pltpu.VMEM((HALF, EMBED), jnp.float32),
            pltpu.VMEM((HALF, EMBED), jnp.float32),
            pltpu.SemaphoreType.DMA,
            pltpu.SemaphoreType.DMA,
        ],
    )
    def emb_kernel(tok_hbm, pidx_hbm, table_hbm, pos_hbm, out_hbm,
                   tok_v, pidx_v, rows_v, pos_rows_v, sem_a, sem_b):
        wid = lax.axis_index("s") * NC + lax.axis_index("c")
        w_base = wid * rows_per_w

        def blk_body(ch, carry):
            base = pl.multiple_of(w_base + ch * BLK, 8)
            pltpu.sync_copy(tok_hbm.at[pl.ds(base, BLK)], tok_v)
            pltpu.sync_copy(pidx_hbm.at[pl.ds(base, BLK)], pidx_v)
            for h in range(2):
                cp1 = pltpu.async_copy(
                    table_hbm.at[tok_v.at[pl.ds(h * HALF, HALF)]],
                    rows_v, sem_a)
                cp2 = pltpu.async_copy(
                    pos_hbm.at[pidx_v.at[pl.ds(h * HALF, HALF)]],
                    pos_rows_v, sem_b)
                cp1.wait()
                cp2.wait()

                @plsc.parallel_loop(0, HALF, unroll=8)
                def _row_body(r):
                    for k in range(EMBED // LANES):
                        sl = pl.ds(k * LANES, LANES)
                        rows_v[r, sl] = rows_v[r, sl] + pos_rows_v[r, sl]

                pltpu.sync_copy(rows_v, out_hbm.at[pl.ds(base + h * HALF, HALF)])
            return carry

        lax.fori_loop(0, nblk, blk_body, 0)

    return emb_kernel


def kernel(x, padding_mask, table, pos_enc):
    b, s = x.shape
    ntok = b * s
    tok = x.reshape(ntok).astype(jnp.int32)
    s_ids = jnp.arange(s, dtype=jnp.int32)[None, :]
    prows = pos_enc.shape[0] + 1
    pidx = jnp.where(padding_mask, jnp.int32(prows - 1), s_ids).reshape(ntok)
    rep = (jnp.arange(ntok, dtype=jnp.int32) % POS_REP) * prows
    pidx = pidx + rep
    pos_ext = jnp.concatenate(
        [pos_enc.astype(jnp.float32),
         jnp.zeros((1, pos_enc.shape[1]), jnp.float32)], axis=0)
    pos_rep = jnp.tile(pos_ext, (POS_REP, 1))
    out = _build(ntok)(tok, pidx, table, pos_rep)
    return out.reshape(b, s, EMBED)


# speedup vs baseline: 5.6532x; 1.0016x over previous
import functools

import jax
import jax.numpy as jnp
from jax import lax
from jax.experimental import pallas as pl
from jax.experimental.pallas import tpu as pltpu
from jax.experimental.pallas import tpu_sc as plsc

EMBED = 64
LANES = 16
NC = 2
NS = 16
NW = NC * NS

STEP = 400
POS_REP = 64


def _build(ntok):
    rows_per_w = ntok // NW
    nsteps = rows_per_w // STEP
    mesh = plsc.VectorSubcoreMesh(core_axis_name="c", subcore_axis_name="s")

    @functools.partial(
        pl.kernel,
        out_type=jax.ShapeDtypeStruct((ntok, EMBED), jnp.float32),
        mesh=mesh,
        compiler_params=pltpu.CompilerParams(use_tc_tiling_on_sc=False),
        scratch_types=[
            pltpu.VMEM((2, STEP), jnp.int32),
            pltpu.VMEM((2, STEP), jnp.int32),
            pltpu.VMEM((STEP, EMBED), jnp.float32),
            pltpu.VMEM((STEP, EMBED), jnp.float32),
            pltpu.VMEM((STEP, EMBED), jnp.float32),
            pltpu.VMEM((STEP, EMBED), jnp.float32),
            pltpu.SemaphoreType.DMA,
            pltpu.SemaphoreType.DMA,
            pltpu.SemaphoreType.DMA,
            pltpu.SemaphoreType.DMA,
            pltpu.SemaphoreType.DMA,
            pltpu.SemaphoreType.DMA,
        ],
    )
    def emb_kernel(tok_hbm, pidx_hbm, table_hbm, pos_hbm, out_hbm,
                   tok_v, pidx_v, rows0, rows1, prows0, prows1,
                   sgt0, sgt1, sgp0, sgp1, so0, so1):
        wid = lax.axis_index("s") * NC + lax.axis_index("c")
        w_base = wid * rows_per_w
        rows = (rows0, rows1)
        prows = (prows0, prows1)
        sgt = (sgt0, sgt1)
        sgp = (sgp0, sgp1)
        so = (so0, so1)

        def issue_gathers(st, b):
            base = pl.multiple_of(w_base + st * STEP, 8)
            pltpu.sync_copy(tok_hbm.at[pl.ds(base, STEP)], tok_v.at[b])
            pltpu.sync_copy(pidx_hbm.at[pl.ds(base, STEP)], pidx_v.at[b])
            pltpu.async_copy(table_hbm.at[tok_v.at[b]], rows[b], sgt[b])
            pltpu.async_copy(pos_hbm.at[pidx_v.at[b]], prows[b], sgp[b])

        def wait_gathers(b):
            pltpu.make_async_copy(table_hbm.at[tok_v.at[b]], rows[b],
                                  sgt[b]).wait()
            pltpu.make_async_copy(pos_hbm.at[pidx_v.at[b]], prows[b],
                                  sgp[b]).wait()

        def wait_out(b):
            pltpu.make_async_copy(rows[b], out_hbm.at[pl.ds(0, STEP)],
                                  so[b]).wait()

        issue_gathers(0, 0)

        def pair_body(j, carry):
            for b in range(2):
                st = 2 * j + b
                nb = 1 - b

                @pl.when(st + 1 < nsteps)
                def _issue_next():
                    @pl.when(st >= 1)
                    def _drain_out():
                        wait_out(nb)
                    issue_gathers(st + 1, nb)

                wait_gathers(b)

                @plsc.parallel_loop(0, STEP, unroll=8)
                def _row_body(r):
                    for k in range(EMBED // LANES):
                        sl = pl.ds(k * LANES, LANES)
                        rows[b][r, sl] = rows[b][r, sl] + prows[b][r, sl]

                base = pl.multiple_of(w_base + st * STEP, 8)
                pltpu.async_copy(rows[b], out_hbm.at[pl.ds(base, STEP)], so[b])
            return carry

        lax.fori_loop(0, nsteps // 2, pair_body, 0)
        wait_out(0)
        wait_out(1)

    return emb_kernel


def kernel(x, padding_mask, table, pos_enc):
    b, s = x.shape
    ntok = b * s
    tok = x.reshape(ntok).astype(jnp.int32)
    s_ids = jnp.arange(s, dtype=jnp.int32)[None, :]
    prows = pos_enc.shape[0] + 1
    pidx = jnp.where(padding_mask, jnp.int32(prows - 1), s_ids).reshape(ntok)
    rep = (jnp.arange(ntok, dtype=jnp.int32) % POS_REP) * prows
    pidx = pidx + rep
    pos_ext = jnp.concatenate(
        [pos_enc.astype(jnp.float32),
         jnp.zeros((1, pos_enc.shape[1]), jnp.float32)], axis=0)
    pos_rep = jnp.tile(pos_ext, (POS_REP, 1))
    out = _build(ntok)(tok, pidx, table, pos_rep)
    return out.reshape(b, s, EMBED)


# speedup vs baseline: 5.6569x; 1.0006x over previous
import functools

import jax
import jax.numpy as jnp
from jax import lax
from jax.experimental import pallas as pl
from jax.experimental.pallas import tpu as pltpu
from jax.experimental.pallas import tpu_sc as plsc

EMBED = 64
LANES = 16
NC = 2
NS = 16
NW = NC * NS

STEP = 400
POS_REP = 64


def _build(ntok):
    rows_per_w = ntok // NW
    nsteps = rows_per_w // STEP
    mesh = plsc.VectorSubcoreMesh(core_axis_name="c", subcore_axis_name="s")

    @functools.partial(
        pl.kernel,
        out_type=jax.ShapeDtypeStruct((ntok, EMBED), jnp.float32),
        mesh=mesh,
        compiler_params=pltpu.CompilerParams(use_tc_tiling_on_sc=False),
        scratch_types=[
            pltpu.VMEM((2, STEP), jnp.int32),
            pltpu.VMEM((2, STEP), jnp.int32),
            pltpu.VMEM((STEP, EMBED), jnp.float32),
            pltpu.VMEM((STEP, EMBED), jnp.float32),
            pltpu.VMEM((STEP, EMBED), jnp.float32),
            pltpu.VMEM((STEP, EMBED), jnp.float32),
            pltpu.SemaphoreType.DMA,
            pltpu.SemaphoreType.DMA,
            pltpu.SemaphoreType.DMA,
            pltpu.SemaphoreType.DMA,
            pltpu.SemaphoreType.DMA,
            pltpu.SemaphoreType.DMA,
        ],
    )
    def emb_kernel(tok_hbm, pidx_hbm, table_hbm, pos_hbm, out_hbm,
                   tok_v, pidx_v, rows0, rows1, prows0, prows1,
                   sgt0, sgt1, sgp0, sgp1, so0, so1):
        wid = lax.axis_index("s") * NC + lax.axis_index("c")
        w_base = wid * rows_per_w
        rows = (rows0, rows1)
        prows = (prows0, prows1)
        sgt = (sgt0, sgt1)
        sgp = (sgp0, sgp1)
        so = (so0, so1)

        H = STEP // 2

        def issue_gathers(st, b):
            base = pl.multiple_of(w_base + st * STEP, 8)
            pltpu.sync_copy(tok_hbm.at[pl.ds(base, STEP)], tok_v.at[b])
            pltpu.sync_copy(pidx_hbm.at[pl.ds(base, STEP)], pidx_v.at[b])
            pltpu.async_copy(table_hbm.at[tok_v.at[b, pl.ds(0, H)]],
                             rows[b].at[pl.ds(0, H)], sgt[b])
            pltpu.async_copy(table_hbm.at[tok_v.at[b, pl.ds(H, H)]],
                             rows[b].at[pl.ds(H, H)], sgt[b])
            pltpu.async_copy(pos_hbm.at[pidx_v.at[b, pl.ds(0, H)]],
                             prows[b].at[pl.ds(0, H)], sgp[b])
            pltpu.async_copy(pos_hbm.at[pidx_v.at[b, pl.ds(H, H)]],
                             prows[b].at[pl.ds(H, H)], sgp[b])

        def wait_gathers(b):
            pltpu.make_async_copy(table_hbm.at[tok_v.at[b]], rows[b],
                                  sgt[b]).wait()
            pltpu.make_async_copy(pos_hbm.at[pidx_v.at[b]], prows[b],
                                  sgp[b]).wait()

        def wait_out(b):
            pltpu.make_async_copy(rows[b], out_hbm.at[pl.ds(0, STEP)],
                                  so[b]).wait()

        issue_gathers(0, 0)

        def pair_body(j, carry):
            for b in range(2):
                st = 2 * j + b
                nb = 1 - b

                @pl.when(st + 1 < nsteps)
                def _issue_next():
                    @pl.when(st >= 1)
                    def _drain_out():
                        wait_out(nb)
                    issue_gathers(st + 1, nb)

                wait_gathers(b)

                @plsc.parallel_loop(0, STEP, unroll=8)
                def _row_body(r):
                    for k in range(EMBED // LANES):
                        sl = pl.ds(k * LANES, LANES)
                        rows[b][r, sl] = rows[b][r, sl] + prows[b][r, sl]

                base = pl.multiple_of(w_base + st * STEP, 8)
                pltpu.async_copy(rows[b], out_hbm.at[pl.ds(base, STEP)], so[b])
            return carry

        lax.fori_loop(0, nsteps // 2, pair_body, 0)
        wait_out(0)
        wait_out(1)

    return emb_kernel


def kernel(x, padding_mask, table, pos_enc):
    b, s = x.shape
    ntok = b * s
    tok = x.reshape(ntok).astype(jnp.int32)
    s_ids = jnp.arange(s, dtype=jnp.int32)[None, :]
    prows = pos_enc.shape[0] + 1
    pidx = jnp.where(padding_mask, jnp.int32(prows - 1), s_ids).reshape(ntok)
    rep = (jnp.arange(ntok, dtype=jnp.int32) % POS_REP) * prows
    pidx = pidx + rep
    pos_ext = jnp.concatenate(
        [pos_enc.astype(jnp.float32),
         jnp.zeros((1, pos_enc.shape[1]), jnp.float32)], axis=0)
    pos_rep = jnp.tile(pos_ext, (POS_REP, 1))
    out = _build(ntok)(tok, pidx, table, pos_rep)
    return out.reshape(b, s, EMBED)
